# Initial kernel scaffold; baseline (speedup 1.0000x reference)
import jax
import jax.numpy as jnp
from jax.experimental import pallas as pl


def kernel(x):
    raise NotImplementedError("write your pallas kernel here")


# speedup vs baseline: 31.4552x; 31.4552x over previous
import functools

import jax
import jax.numpy as jnp
import numpy as np
from jax.experimental import pallas as pl
from jax.experimental.pallas import tpu as pltpu

VOL_X, VOL_Y = 512, 512
N_DET = 736
N_ANGLES = 720
BATCH = 8
D_PAD = 768
N_CHUNK = D_PAD // 128
DET_CENTER = (N_DET - 1) / 2.0
ANG_BLK = 8
X_HALF = VOL_X // 2


def _bp_kernel(sc_ref, sino_ref, out_ref):
    h = pl.program_id(0)
    c = pl.program_id(1)

    @pl.when(c == 0)
    def _init():
        out_ref[...] = jnp.zeros((X_HALF, BATCH, VOL_Y), jnp.float32)

    ys = jax.lax.broadcasted_iota(jnp.int32, (BATCH, VOL_Y), 1).astype(jnp.float32) - (VOL_Y - 1) / 2.0
    x_off = (h * X_HALF).astype(jnp.float32) - (VOL_X - 1) / 2.0

    for k in range(ANG_BLK):
        a = c * ANG_BLK + k
        cos_a = sc_ref[a, 0]
        sin_a = sc_ref[a, 1]
        ys_s = ys * sin_a

        def body(i, _, k=k, cos_a=cos_a, sin_a=sin_a, ys_s=ys_s):
            base = (x_off + i.astype(jnp.float32)) * cos_a + DET_CENTER
            acc = out_ref[i]
            contribs = []
            for t in range(4):
                q = jnp.float32(128.0 * t - (VOL_Y - 1) / 2.0) * sin_a + base
                qt = q.astype(jnp.int32)
                pbase = jnp.clip(qt, 0, N_DET - 97) & ~jnp.int32(127)
                p = pbase >> 7
                v0 = sino_ref[k, p]
                v1 = sino_ref[k, p + 1]

                u = ys_s[:, 128 * t:128 * (t + 1)] + base
                u_t = jnp.trunc(u)
                w = u - u_t
                idx = u_t.astype(jnp.int32) - pbase
                valid = (u >= 0.0) & (u <= N_DET - 1.0)
                w1 = jnp.where(valid, w, 0.0)
                w0 = jnp.where(valid, 1.0 - w, 0.0)

                idxm = idx & 127
                g0 = jnp.where(idx < 128,
                               jnp.take_along_axis(v0, idxm, axis=1),
                               jnp.take_along_axis(v1, idxm, axis=1))
                idx1 = idx + 1
                idx1m = idx1 & 127
                g1 = jnp.where(idx1 < 128,
                               jnp.take_along_axis(v0, idx1m, axis=1),
                               jnp.take_along_axis(v1, idx1m, axis=1))
                contribs.append(g0 * w0 + g1 * w1)
            out_ref[i] = acc + jnp.concatenate(contribs, axis=-1)
            return _

        jax.lax.fori_loop(0, X_HALF, body, None)


@jax.jit
def kernel(x):
    sino = jnp.transpose(x[..., 0], (1, 0, 2))
    sino = jnp.pad(sino, ((0, 0), (0, 0), (0, D_PAD - N_DET)))
    sino = sino.reshape(N_ANGLES, BATCH, N_CHUNK, 128).transpose(0, 2, 1, 3)

    th = np.arange(N_ANGLES, dtype=np.float64) * (np.pi / N_ANGLES)
    sc = jnp.asarray(np.stack([np.cos(th), np.sin(th)], axis=1).astype(np.float32))

    grid_spec = pltpu.PrefetchScalarGridSpec(
        num_scalar_prefetch=1,
        grid=(2, N_ANGLES // ANG_BLK),
        in_specs=[
            pl.BlockSpec((ANG_BLK, N_CHUNK, BATCH, 128), lambda h, c, sc: (c, 0, 0, 0)),
        ],
        out_specs=pl.BlockSpec((X_HALF, BATCH, VOL_Y), lambda h, c, sc: (h, 0, 0)),
    )
    vol = pl.pallas_call(
        _bp_kernel,
        grid_spec=grid_spec,
        out_shape=jax.ShapeDtypeStruct((VOL_X, BATCH, VOL_Y), jnp.float32),
        compiler_params=pltpu.CompilerParams(
            dimension_semantics=("parallel", "arbitrary"),
        ),
    )(sc, sino)

    vol = jnp.transpose(vol, (1, 0, 2)) * jnp.float32(np.pi / N_ANGLES)
    return vol[..., None]


# speedup vs baseline: 71.3738x; 2.2691x over previous
import jax
import jax.numpy as jnp
import numpy as np
from jax.experimental import pallas as pl
from jax.experimental.pallas import tpu as pltpu

VOL_X, VOL_Y = 512, 512
N_DET = 736
N_ANGLES = 720
BATCH = 8
D_PAD = 768
N_CHUNK = D_PAD // 128
DET_CENTER = (N_DET - 1) / 2.0
ANG_BLK = 8
X_HALF = VOL_X // 2
ROW_UNROLL = 4


def _bp_kernel(sc_ref, sino_ref, sino_s_ref, out_ref):
    h = pl.program_id(0)
    c = pl.program_id(1)

    @pl.when(c == 0)
    def _init():
        out_ref[...] = jnp.zeros((X_HALF, BATCH, VOL_Y), jnp.float32)

    ys = jax.lax.broadcasted_iota(jnp.int32, (BATCH, VOL_Y), 1).astype(jnp.float32) - (VOL_Y - 1) / 2.0
    x_off = (h * X_HALF).astype(jnp.float32) - (VOL_X - 1) / 2.0

    for k in range(ANG_BLK):
        a = c * ANG_BLK + k
        cos_a = sc_ref[a, 0]
        sin_a = sc_ref[a, 1]
        ys_s = ys * sin_a

        def body(iu, _, k=k, cos_a=cos_a, sin_a=sin_a, ys_s=ys_s):
            accs = [[out_ref[iu * ROW_UNROLL + r, :, 128 * t:128 * (t + 1)]
                     for t in range(4)] for r in range(ROW_UNROLL)]
            for r in range(ROW_UNROLL):
                i = iu * ROW_UNROLL + r
                base = (x_off + i.astype(jnp.float32)) * cos_a + DET_CENTER
                for t in range(4):
                    q = jnp.float32(128.0 * t - (VOL_Y - 1) / 2.0) * sin_a + base
                    qt = q.astype(jnp.int32)
                    pbase = jnp.clip(qt, 0, N_DET - 97) & ~jnp.int32(127)
                    p = pbase >> 7
                    v0 = sino_ref[k, p]
                    v1 = sino_ref[k, p + 1]
                    v0s = sino_s_ref[k, p]
                    v1s = sino_s_ref[k, p + 1]

                    u = ys_s[:, 128 * t:128 * (t + 1)] + base
                    u_t = jnp.trunc(u)
                    w = u - u_t
                    idx = u_t.astype(jnp.int32) - pbase
                    valid = (u >= 0.0) & (u <= N_DET - 1.0)
                    w1 = jnp.where(valid, w, 0.0)
                    w0 = jnp.where(valid, 1.0 - w, 0.0)

                    idxm = idx & 127
                    m = idx < 128
                    g0 = jnp.where(m,
                                   jnp.take_along_axis(v0, idxm, axis=1),
                                   jnp.take_along_axis(v1, idxm, axis=1))
                    g1 = jnp.where(m,
                                   jnp.take_along_axis(v0s, idxm, axis=1),
                                   jnp.take_along_axis(v1s, idxm, axis=1))
                    accs[r][t] = accs[r][t] + g0 * w0 + g1 * w1
            for r in range(ROW_UNROLL):
                for t in range(4):
                    out_ref[iu * ROW_UNROLL + r, :, 128 * t:128 * (t + 1)] = accs[r][t]
            return _

        jax.lax.fori_loop(0, X_HALF // ROW_UNROLL, body, None)


@jax.jit
def kernel(x):
    sino = jnp.transpose(x[..., 0], (1, 0, 2))
    sino_p = jnp.pad(sino, ((0, 0), (0, 0), (0, D_PAD - N_DET)))
    sino_s = jnp.pad(sino[:, :, 1:], ((0, 0), (0, 0), (0, D_PAD - N_DET + 1)))
    sino_p = sino_p.reshape(N_ANGLES, BATCH, N_CHUNK, 128).transpose(0, 2, 1, 3)
    sino_s = sino_s.reshape(N_ANGLES, BATCH, N_CHUNK, 128).transpose(0, 2, 1, 3)

    th = np.arange(N_ANGLES, dtype=np.float64) * (np.pi / N_ANGLES)
    sc = jnp.asarray(np.stack([np.cos(th), np.sin(th)], axis=1).astype(np.float32))

    spec = pl.BlockSpec((ANG_BLK, N_CHUNK, BATCH, 128), lambda h, c, sc: (c, 0, 0, 0))
    grid_spec = pltpu.PrefetchScalarGridSpec(
        num_scalar_prefetch=1,
        grid=(2, N_ANGLES // ANG_BLK),
        in_specs=[spec, spec],
        out_specs=pl.BlockSpec((X_HALF, BATCH, VOL_Y), lambda h, c, sc: (h, 0, 0)),
    )
    vol = pl.pallas_call(
        _bp_kernel,
        grid_spec=grid_spec,
        out_shape=jax.ShapeDtypeStruct((VOL_X, BATCH, VOL_Y), jnp.float32),
        compiler_params=pltpu.CompilerParams(
            dimension_semantics=("parallel", "arbitrary"),
        ),
    )(sc, sino_p, sino_s)

    vol = jnp.transpose(vol, (1, 0, 2)) * jnp.float32(np.pi / N_ANGLES)
    return vol[..., None]


# speedup vs baseline: 91.4006x; 1.2806x over previous
import jax
import jax.numpy as jnp
import numpy as np
from jax.experimental import pallas as pl
from jax.experimental.pallas import tpu as pltpu

VOL_X, VOL_Y = 512, 512
N_DET = 736
N_ANGLES = 720
BATCH = 8
D_PAD = 768
N_CHUNK = D_PAD // 128
DET_CENTER = (N_DET - 1) / 2.0
ANG_BLK = 8
X_HALF = VOL_X // 2
ROW_UNROLL = 4


def _bp_kernel(sc_ref, sino_ref, sino_s_ref, out_ref):
    h = pl.program_id(0)
    c = pl.program_id(1)

    @pl.when(c == 0)
    def _init():
        out_ref[...] = jnp.zeros((X_HALF, BATCH, VOL_Y), jnp.float32)

    ys = jax.lax.broadcasted_iota(jnp.int32, (BATCH, VOL_Y), 1).astype(jnp.float32) - (VOL_Y - 1) / 2.0
    x_off = (h * X_HALF).astype(jnp.float32) - (VOL_X - 1) / 2.0

    cs = [(sc_ref[c * ANG_BLK + k, 0], sc_ref[c * ANG_BLK + k, 1])
          for k in range(ANG_BLK)]

    def body(i, _):
        fi = x_off + i.astype(jnp.float32)
        bases = [fi * cos_a + DET_CENTER for cos_a, _ in cs]
        for t in range(4):
            ys_t = ys[:, 128 * t:128 * (t + 1)]
            acc = out_ref[i, :, 128 * t:128 * (t + 1)]
            contribs = []
            for k in range(ANG_BLK):
                cos_a, sin_a = cs[k]
                base = bases[k]
                q = jnp.float32(128.0 * t - (VOL_Y - 1) / 2.0) * sin_a + base
                qt = q.astype(jnp.int32)
                pbase = jnp.clip(qt, 0, N_DET - 97) & ~jnp.int32(127)
                p = pbase >> 7
                v0 = sino_ref[k, p]
                v1 = sino_ref[k, p + 1]
                v0s = sino_s_ref[k, p]
                v1s = sino_s_ref[k, p + 1]

                u = ys_t * sin_a + base
                u_t = jnp.trunc(u)
                w = u - u_t
                idx = u_t.astype(jnp.int32) - pbase
                valid = (u >= 0.0) & (u <= N_DET - 1.0)
                w1 = jnp.where(valid, w, 0.0)
                w0 = jnp.where(valid, 1.0 - w, 0.0)

                idxm = idx & 127
                m = idx < 128
                g0 = jnp.where(m,
                               jnp.take_along_axis(v0, idxm, axis=1),
                               jnp.take_along_axis(v1, idxm, axis=1))
                g1 = jnp.where(m,
                               jnp.take_along_axis(v0s, idxm, axis=1),
                               jnp.take_along_axis(v1s, idxm, axis=1))
                contribs.append(g0 * w0 + g1 * w1)
            while len(contribs) > 1:
                contribs = [contribs[j] + contribs[j + 1]
                            for j in range(0, len(contribs) - 1, 2)] + (
                    [contribs[-1]] if len(contribs) % 2 else [])
            out_ref[i, :, 128 * t:128 * (t + 1)] = acc + contribs[0]
        return _

    jax.lax.fori_loop(0, X_HALF, body, None)


@jax.jit
def kernel(x):
    sino = jnp.transpose(x[..., 0], (1, 0, 2))
    sino_p = jnp.pad(sino, ((0, 0), (0, 0), (0, D_PAD - N_DET)))
    sino_s = jnp.pad(sino[:, :, 1:], ((0, 0), (0, 0), (0, D_PAD - N_DET + 1)))
    sino_p = sino_p.reshape(N_ANGLES, BATCH, N_CHUNK, 128).transpose(0, 2, 1, 3)
    sino_s = sino_s.reshape(N_ANGLES, BATCH, N_CHUNK, 128).transpose(0, 2, 1, 3)

    th = np.arange(N_ANGLES, dtype=np.float64) * (np.pi / N_ANGLES)
    sc = jnp.asarray(np.stack([np.cos(th), np.sin(th)], axis=1).astype(np.float32))

    spec = pl.BlockSpec((ANG_BLK, N_CHUNK, BATCH, 128), lambda h, c, sc: (c, 0, 0, 0))
    grid_spec = pltpu.PrefetchScalarGridSpec(
        num_scalar_prefetch=1,
        grid=(2, N_ANGLES // ANG_BLK),
        in_specs=[spec, spec],
        out_specs=pl.BlockSpec((X_HALF, BATCH, VOL_Y), lambda h, c, sc: (h, 0, 0)),
    )
    vol = pl.pallas_call(
        _bp_kernel,
        grid_spec=grid_spec,
        out_shape=jax.ShapeDtypeStruct((VOL_X, BATCH, VOL_Y), jnp.float32),
        compiler_params=pltpu.CompilerParams(
            dimension_semantics=("parallel", "arbitrary"),
        ),
    )(sc, sino_p, sino_s)

    vol = jnp.transpose(vol, (1, 0, 2)) * jnp.float32(np.pi / N_ANGLES)
    return vol[..., None]


# speedup vs baseline: 118.7488x; 1.2992x over previous
import jax
import jax.numpy as jnp
import numpy as np
from jax.experimental import pallas as pl
from jax.experimental.pallas import tpu as pltpu

VOL_X, VOL_Y = 512, 512
N_DET = 736
N_ANGLES = 720
BATCH = 8
D_PAD = 768
N_CHUNK = D_PAD // 128
DET_CENTER = (N_DET - 1) / 2.0
ANG_BLK = 8
X_HALF = VOL_X // 2
N_IG = X_HALF // 8


def _bp_kernel(sc_ref, sino_ref, out_ref, tab_ref, tabs_ref):
    h = pl.program_id(0)
    c = pl.program_id(1)

    @pl.when(c == 0)
    def _init():
        out_ref[...] = jnp.zeros((BATCH, X_HALF, VOL_Y), jnp.float32)

    for k in range(ANG_BLK):
        for ch in range(N_CHUNK):
            src = sino_ref[k, ch]
            if ch < N_CHUNK - 1:
                nxt = sino_ref[k, ch + 1][:, :1]
            else:
                nxt = jnp.zeros((BATCH, 1), jnp.float32)
            srcs = jnp.concatenate([src[:, 1:], nxt], axis=-1)
            for b in range(BATCH):
                tab_ref[k, b, ch] = jnp.broadcast_to(src[b:b + 1], (8, 128))
                tabs_ref[k, b, ch] = jnp.broadcast_to(srcs[b:b + 1], (8, 128))

    cs = [(sc_ref[c * ANG_BLK + k, 0], sc_ref[c * ANG_BLK + k, 1])
          for k in range(ANG_BLK)]
    corner = [jnp.where(cos_a < 0.0, 7.0, 0.0).astype(jnp.float32)
              for cos_a, _ in cs]

    x_off = (h * X_HALF).astype(jnp.float32) - (VOL_X - 1) / 2.0
    xsv = jax.lax.broadcasted_iota(jnp.int32, (8, 128), 0).astype(jnp.float32) + x_off
    ysl = jax.lax.broadcasted_iota(jnp.int32, (8, 128), 1).astype(jnp.float32)

    def body(ig, _):
        ib = pl.multiple_of(ig * 8, 8)
        ig8 = (ig * 8).astype(jnp.float32)
        xsc = xsv + ig8
        for t in range(4):
            ct = jnp.float32(128.0 * t - (VOL_Y - 1) / 2.0)
            gathered = []
            for k in range(ANG_BLK):
                cos_a, sin_a = cs[k]
                q = ((corner[k] + x_off) + ig8) * cos_a + (ct * sin_a + DET_CENTER)
                qt = q.astype(jnp.int32)
                pbase = jnp.clip(qt, 0, N_DET - 97) & ~jnp.int32(127)
                p = pbase >> 7
                pb128 = (pbase + 128).astype(jnp.float32)

                u = xsc * cos_a + ((ysl + ct) * sin_a + DET_CENTER)
                u_t = jnp.trunc(u)
                w = u - u_t
                idxm = u_t.astype(jnp.int32) & 127
                m = u < pb128
                valid = (u >= 0.0) & (u <= N_DET - 1.0)
                w1 = jnp.where(valid, w, 0.0)
                w0 = jnp.where(valid, 1.0 - w, 0.0)
                gathered.append((idxm, m, w0, w1, p))

            for b in range(BATCH):
                acc = out_ref[b, pl.ds(ib, 8), 128 * t:128 * (t + 1)]
                contribs = []
                for k in range(ANG_BLK):
                    idxm, m, w0, w1, p = gathered[k]
                    g0 = jnp.where(m,
                                   jnp.take_along_axis(tab_ref[k, b, p], idxm, axis=1),
                                   jnp.take_along_axis(tab_ref[k, b, p + 1], idxm, axis=1))
                    g1 = jnp.where(m,
                                   jnp.take_along_axis(tabs_ref[k, b, p], idxm, axis=1),
                                   jnp.take_along_axis(tabs_ref[k, b, p + 1], idxm, axis=1))
                    contribs.append(g0 * w0 + g1 * w1)
                while len(contribs) > 1:
                    contribs = [contribs[j] + contribs[j + 1]
                                for j in range(0, len(contribs) - 1, 2)] + (
                        [contribs[-1]] if len(contribs) % 2 else [])
                out_ref[b, pl.ds(ib, 8), 128 * t:128 * (t + 1)] = acc + contribs[0]
        return _

    jax.lax.fori_loop(0, N_IG, body, None)


@jax.jit
def kernel(x):
    sino = jnp.transpose(x[..., 0], (1, 0, 2))
    sino_p = jnp.pad(sino, ((0, 0), (0, 0), (0, D_PAD - N_DET)))
    sino_p = sino_p.reshape(N_ANGLES, BATCH, N_CHUNK, 128).transpose(0, 2, 1, 3)

    th = np.arange(N_ANGLES, dtype=np.float64) * (np.pi / N_ANGLES)
    sc = jnp.asarray(np.stack([np.cos(th), np.sin(th)], axis=1).astype(np.float32))

    grid_spec = pltpu.PrefetchScalarGridSpec(
        num_scalar_prefetch=1,
        grid=(2, N_ANGLES // ANG_BLK),
        in_specs=[
            pl.BlockSpec((ANG_BLK, N_CHUNK, BATCH, 128), lambda h, c, sc: (c, 0, 0, 0)),
        ],
        out_specs=pl.BlockSpec((BATCH, X_HALF, VOL_Y), lambda h, c, sc: (0, h, 0)),
        scratch_shapes=[
            pltpu.VMEM((ANG_BLK, BATCH, N_CHUNK, 8, 128), jnp.float32),
            pltpu.VMEM((ANG_BLK, BATCH, N_CHUNK, 8, 128), jnp.float32),
        ],
    )
    vol = pl.pallas_call(
        _bp_kernel,
        grid_spec=grid_spec,
        out_shape=jax.ShapeDtypeStruct((BATCH, VOL_X, VOL_Y), jnp.float32),
        compiler_params=pltpu.CompilerParams(
            dimension_semantics=("parallel", "arbitrary"),
        ),
    )(sc, sino_p)

    return vol[..., None] * jnp.float32(np.pi / N_ANGLES)


# speedup vs baseline: 179.7425x; 1.5136x over previous
import jax
import jax.numpy as jnp
import numpy as np
from jax.experimental import pallas as pl
from jax.experimental.pallas import tpu as pltpu

VOL_X, VOL_Y = 512, 512
N_DET = 736
N_ANGLES = 720
BATCH = 8
D_PAD = 768
N_CHUNK = D_PAD // 128
DET_CENTER = (N_DET - 1) / 2.0
ANG_BLK = 8
X_HALF = VOL_X // 2
N_IG = X_HALF // 8


def _bp_kernel(sc_ref, sino_ref, out_ref, tab_ref, tabs_ref):
    h = pl.program_id(0)
    c = pl.program_id(1)

    @pl.when(c == 0)
    def _init():
        out_ref[...] = jnp.zeros((BATCH, X_HALF, VOL_Y), jnp.float32)

    for k in range(ANG_BLK):
        for ch in range(N_CHUNK):
            src = sino_ref[k, ch]
            if ch < N_CHUNK - 1:
                nxt = sino_ref[k, ch + 1][:, :1]
            else:
                nxt = jnp.zeros((BATCH, 1), jnp.float32)
            srcs = jnp.concatenate([src[:, 1:], nxt], axis=-1)
            for b in range(BATCH):
                tab_ref[k, b, ch] = jnp.broadcast_to(src[b:b + 1], (8, 128))
                tabs_ref[k, b, ch] = jnp.broadcast_to(srcs[b:b + 1], (8, 128))

    cs = [(sc_ref[c * ANG_BLK + k, 0], sc_ref[c * ANG_BLK + k, 1])
          for k in range(ANG_BLK)]
    corner = [jnp.where(cos_a < 0.0, 7.0, 0.0).astype(jnp.float32)
              for cos_a, _ in cs]

    x_off = (h * X_HALF).astype(jnp.float32) - (VOL_X - 1) / 2.0
    xsv = jax.lax.broadcasted_iota(jnp.int32, (8, 128), 0).astype(jnp.float32) + x_off
    ysl = jax.lax.broadcasted_iota(jnp.int32, (8, 128), 1).astype(jnp.float32)

    def body(ig, _):
        ib = pl.multiple_of(ig * 8, 8)
        ig8 = (ig * 8).astype(jnp.float32)
        xsc = xsv + ig8
        for t in range(4):
            ct = jnp.float32(128.0 * t - (VOL_Y - 1) / 2.0)
            gathered = []
            for k in range(ANG_BLK):
                cos_a, sin_a = cs[k]
                q = ((corner[k] + x_off) + ig8) * cos_a + (ct * sin_a + DET_CENTER)
                qt = q.astype(jnp.int32)
                pbase = jnp.clip(qt, 0, N_DET - 97) & ~jnp.int32(127)
                p = pbase >> 7
                pb128 = (pbase + 128).astype(jnp.float32)

                u = xsc * cos_a + ((ysl + ct) * sin_a + DET_CENTER)
                u_t = jnp.trunc(u)
                w = u - u_t
                idxm = u_t.astype(jnp.int32) & 127
                m = u < pb128
                valid = (u >= 0.0) & (u <= N_DET - 1.0)
                w1 = jnp.where(valid, w, 0.0)
                w0 = jnp.where(valid, 1.0 - w, 0.0)
                gathered.append((idxm, m, w0, w1, p))

            accs = [out_ref[b, pl.ds(ib, 8), 128 * t:128 * (t + 1)]
                    for b in range(BATCH)]
            for k in range(ANG_BLK):
                idxm, m, w0, w1, p = gathered[k]
                for b in range(BATCH):
                    g0 = jnp.where(m,
                                   jnp.take_along_axis(tab_ref[k, b, p], idxm, axis=1),
                                   jnp.take_along_axis(tab_ref[k, b, p + 1], idxm, axis=1))
                    g1 = jnp.where(m,
                                   jnp.take_along_axis(tabs_ref[k, b, p], idxm, axis=1),
                                   jnp.take_along_axis(tabs_ref[k, b, p + 1], idxm, axis=1))
                    accs[b] = accs[b] + (g0 * w0 + g1 * w1)
            for b in range(BATCH):
                out_ref[b, pl.ds(ib, 8), 128 * t:128 * (t + 1)] = accs[b]
        return _

    jax.lax.fori_loop(0, N_IG, body, None)


@jax.jit
def kernel(x):
    sino = jnp.transpose(x[..., 0], (1, 0, 2))
    sino_p = jnp.pad(sino, ((0, 0), (0, 0), (0, D_PAD - N_DET)))
    sino_p = sino_p.reshape(N_ANGLES, BATCH, N_CHUNK, 128).transpose(0, 2, 1, 3)

    th = np.arange(N_ANGLES, dtype=np.float64) * (np.pi / N_ANGLES)
    sc = jnp.asarray(np.stack([np.cos(th), np.sin(th)], axis=1).astype(np.float32))

    grid_spec = pltpu.PrefetchScalarGridSpec(
        num_scalar_prefetch=1,
        grid=(2, N_ANGLES // ANG_BLK),
        in_specs=[
            pl.BlockSpec((ANG_BLK, N_CHUNK, BATCH, 128), lambda h, c, sc: (c, 0, 0, 0)),
        ],
        out_specs=pl.BlockSpec((BATCH, X_HALF, VOL_Y), lambda h, c, sc: (0, h, 0)),
        scratch_shapes=[
            pltpu.VMEM((ANG_BLK, BATCH, N_CHUNK, 8, 128), jnp.float32),
            pltpu.VMEM((ANG_BLK, BATCH, N_CHUNK, 8, 128), jnp.float32),
        ],
    )
    vol = pl.pallas_call(
        _bp_kernel,
        grid_spec=grid_spec,
        out_shape=jax.ShapeDtypeStruct((BATCH, VOL_X, VOL_Y), jnp.float32),
        compiler_params=pltpu.CompilerParams(
            dimension_semantics=("parallel", "arbitrary"),
        ),
    )(sc, sino_p)

    return vol[..., None] * jnp.float32(np.pi / N_ANGLES)


# speedup vs baseline: 313.1378x; 1.7421x over previous
import jax
import jax.numpy as jnp
import numpy as np
from jax.experimental import pallas as pl
from jax.experimental.pallas import tpu as pltpu

VOL_X, VOL_Y = 512, 512
N_DET = 736
N_ANGLES = 720
BATCH = 8
D_PAD = 768
N_CHUNK = D_PAD // 128
DET_CENTER = (N_DET - 1) / 2.0
ANG_BLK = 8
X_HALF = VOL_X // 2
N_IG = X_HALF // 8


def _bp_kernel(sc_ref, sino_ref, out_ref, tab_ref):
    h = pl.program_id(0)
    c = pl.program_id(1)

    @pl.when(c == 0)
    def _init():
        out_ref[...] = jnp.zeros((BATCH, X_HALF, VOL_Y), jnp.float32)

    for k in range(ANG_BLK):
        for ch in range(N_CHUNK):
            src = sino_ref[k, ch]
            if ch < N_CHUNK - 1:
                nxt = sino_ref[k, ch + 1][:, :1]
            else:
                nxt = jnp.zeros((BATCH, 1), jnp.float32)
            srcs = jnp.concatenate([src[:, 1:], nxt], axis=-1)
            packed = pltpu.pack_elementwise([src, srcs], packed_dtype=jnp.bfloat16)
            for b in range(BATCH):
                tab_ref[k, b, ch] = jnp.broadcast_to(packed[b:b + 1], (8, 128))

    cs = [(sc_ref[c * ANG_BLK + k, 0], sc_ref[c * ANG_BLK + k, 1])
          for k in range(ANG_BLK)]
    corner = [jnp.where(cos_a < 0.0, 7.0, 0.0).astype(jnp.float32)
              for cos_a, _ in cs]

    x_off = (h * X_HALF).astype(jnp.float32) - (VOL_X - 1) / 2.0
    xsv = jax.lax.broadcasted_iota(jnp.int32, (8, 128), 0).astype(jnp.float32) + x_off
    ysl = jax.lax.broadcasted_iota(jnp.int32, (8, 128), 1).astype(jnp.float32)

    def body(ig, _):
        ib = pl.multiple_of(ig * 8, 8)
        ig8 = (ig * 8).astype(jnp.float32)
        xsc = xsv + ig8
        for t in range(4):
            ct = jnp.float32(128.0 * t - (VOL_Y - 1) / 2.0)
            gathered = []
            for k in range(ANG_BLK):
                cos_a, sin_a = cs[k]
                q = ((corner[k] + x_off) + ig8) * cos_a + (ct * sin_a + DET_CENTER)
                qt = q.astype(jnp.int32)
                pbase = jnp.clip(qt, 0, N_DET - 97) & ~jnp.int32(127)
                p = pbase >> 7
                pb128 = (pbase + 128).astype(jnp.float32)

                u = xsc * cos_a + ((ysl + ct) * sin_a + DET_CENTER)
                u_t = jnp.trunc(u)
                w = u - u_t
                idxm = u_t.astype(jnp.int32) & 127
                m = u < pb128
                valid = (u >= 0.0) & (u <= N_DET - 1.0)
                w1 = jnp.where(valid, w, 0.0)
                w0 = jnp.where(valid, 1.0 - w, 0.0)
                gathered.append((idxm, m, w0, w1, p))

            accs = [out_ref[b, pl.ds(ib, 8), 128 * t:128 * (t + 1)]
                    for b in range(BATCH)]
            for k in range(ANG_BLK):
                idxm, m, w0, w1, p = gathered[k]
                for b in range(BATCH):
                    word = jnp.where(m,
                                     jnp.take_along_axis(tab_ref[k, b, p], idxm, axis=1),
                                     jnp.take_along_axis(tab_ref[k, b, p + 1], idxm, axis=1))
                    g0 = pltpu.unpack_elementwise(word, index=0,
                                                  packed_dtype=jnp.bfloat16,
                                                  unpacked_dtype=jnp.float32)
                    g1 = pltpu.unpack_elementwise(word, index=1,
                                                  packed_dtype=jnp.bfloat16,
                                                  unpacked_dtype=jnp.float32)
                    accs[b] = accs[b] + (g0 * w0 + g1 * w1)
            for b in range(BATCH):
                out_ref[b, pl.ds(ib, 8), 128 * t:128 * (t + 1)] = accs[b]
        return _

    jax.lax.fori_loop(0, N_IG, body, None)


@jax.jit
def kernel(x):
    sino = jnp.transpose(x[..., 0], (1, 0, 2))
    sino_p = jnp.pad(sino, ((0, 0), (0, 0), (0, D_PAD - N_DET)))
    sino_p = sino_p.reshape(N_ANGLES, BATCH, N_CHUNK, 128).transpose(0, 2, 1, 3)

    th = np.arange(N_ANGLES, dtype=np.float64) * (np.pi / N_ANGLES)
    sc = jnp.asarray(np.stack([np.cos(th), np.sin(th)], axis=1).astype(np.float32))

    grid_spec = pltpu.PrefetchScalarGridSpec(
        num_scalar_prefetch=1,
        grid=(2, N_ANGLES // ANG_BLK),
        in_specs=[
            pl.BlockSpec((ANG_BLK, N_CHUNK, BATCH, 128), lambda h, c, sc: (c, 0, 0, 0)),
        ],
        out_specs=pl.BlockSpec((BATCH, X_HALF, VOL_Y), lambda h, c, sc: (0, h, 0)),
        scratch_shapes=[
            pltpu.VMEM((ANG_BLK, BATCH, N_CHUNK, 8, 128), jnp.int32),
        ],
    )
    vol = pl.pallas_call(
        _bp_kernel,
        grid_spec=grid_spec,
        out_shape=jax.ShapeDtypeStruct((BATCH, VOL_X, VOL_Y), jnp.float32),
        compiler_params=pltpu.CompilerParams(
            dimension_semantics=("parallel", "arbitrary"),
        ),
    )(sc, sino_p)

    return vol[..., None] * jnp.float32(np.pi / N_ANGLES)


# speedup vs baseline: 313.1965x; 1.0002x over previous
import jax
import jax.numpy as jnp
import numpy as np
from jax.experimental import pallas as pl
from jax.experimental.pallas import tpu as pltpu

VOL_X, VOL_Y = 512, 512
N_DET = 736
N_ANGLES = 720
BATCH = 8
D_PAD = 768
N_CHUNK = D_PAD // 128
DET_CENTER = (N_DET - 1) / 2.0
ANG_BLK = 8
X_HALF = VOL_X // 2
N_IG = X_HALF // 8


def _bp_kernel(sc_ref, sino_ref, o0_ref, o1_ref, o2_ref, o3_ref, tab_ref):
    h = pl.program_id(0)
    c = pl.program_id(1)
    out_refs = (o0_ref, o1_ref, o2_ref, o3_ref)

    @pl.when(c == 0)
    def _init():
        for o in out_refs:
            o[...] = jnp.zeros((BATCH, X_HALF, 128), jnp.float32)

    for k in range(ANG_BLK):
        for ch in range(N_CHUNK):
            src = sino_ref[k, ch]
            if ch < N_CHUNK - 1:
                nxt = sino_ref[k, ch + 1][:, :1]
            else:
                nxt = jnp.zeros((BATCH, 1), jnp.float32)
            srcs = jnp.concatenate([src[:, 1:], nxt], axis=-1)
            packed = pltpu.pack_elementwise([src, srcs], packed_dtype=jnp.bfloat16)
            for b in range(BATCH):
                tab_ref[k, b, ch] = jnp.broadcast_to(packed[b:b + 1], (8, 128))

    cs = [(sc_ref[c * ANG_BLK + k, 0], sc_ref[c * ANG_BLK + k, 1])
          for k in range(ANG_BLK)]
    corner = [jnp.where(cos_a < 0.0, 7.0, 0.0).astype(jnp.float32)
              for cos_a, _ in cs]

    x_off = (h * X_HALF).astype(jnp.float32) - (VOL_X - 1) / 2.0
    xsv = jax.lax.broadcasted_iota(jnp.int32, (8, 128), 0).astype(jnp.float32) + x_off
    ysl = jax.lax.broadcasted_iota(jnp.int32, (8, 128), 1).astype(jnp.float32)

    def body(ig, _):
        ib = pl.multiple_of(ig * 8, 8)
        ig8 = (ig * 8).astype(jnp.float32)
        xsc = xsv + ig8
        for t in range(4):
            ct = jnp.float32(128.0 * t - (VOL_Y - 1) / 2.0)
            gathered = []
            for k in range(ANG_BLK):
                cos_a, sin_a = cs[k]
                q = ((corner[k] + x_off) + ig8) * cos_a + (ct * sin_a + DET_CENTER)
                qt = q.astype(jnp.int32)
                pbase = jnp.clip(qt, 0, N_DET - 97) & ~jnp.int32(127)
                p = pbase >> 7
                pb128 = (pbase + 128).astype(jnp.float32)

                u = xsc * cos_a + ((ysl + ct) * sin_a + DET_CENTER)
                u_t = jnp.trunc(u)
                w = u - u_t
                idxm = u_t.astype(jnp.int32) & 127
                m = u < pb128
                valid = (u >= 0.0) & (u <= N_DET - 1.0)
                w1 = jnp.where(valid, w, 0.0)
                w0 = jnp.where(valid, 1.0 - w, 0.0)
                gathered.append((idxm, m, w0, w1, p))

            accs = [out_refs[t][b, pl.ds(ib, 8), :]
                    for b in range(BATCH)]
            for k in range(ANG_BLK):
                idxm, m, w0, w1, p = gathered[k]
                for b in range(BATCH):
                    word = jnp.where(m,
                                     jnp.take_along_axis(tab_ref[k, b, p], idxm, axis=1),
                                     jnp.take_along_axis(tab_ref[k, b, p + 1], idxm, axis=1))
                    g0 = pltpu.unpack_elementwise(word, index=0,
                                                  packed_dtype=jnp.bfloat16,
                                                  unpacked_dtype=jnp.float32)
                    g1 = pltpu.unpack_elementwise(word, index=1,
                                                  packed_dtype=jnp.bfloat16,
                                                  unpacked_dtype=jnp.float32)
                    accs[b] = accs[b] + (g0 * w0 + g1 * w1)
            for b in range(BATCH):
                out_refs[t][b, pl.ds(ib, 8), :] = accs[b]
        return _

    jax.lax.fori_loop(0, N_IG, body, None)


@jax.jit
def kernel(x):
    sino = jnp.transpose(x[..., 0], (1, 0, 2))
    sino_p = jnp.pad(sino, ((0, 0), (0, 0), (0, D_PAD - N_DET)))
    sino_p = sino_p.reshape(N_ANGLES, BATCH, N_CHUNK, 128).transpose(0, 2, 1, 3)

    th = np.arange(N_ANGLES, dtype=np.float64) * (np.pi / N_ANGLES)
    sc = jnp.asarray(np.stack([np.cos(th), np.sin(th)], axis=1).astype(np.float32))

    grid_spec = pltpu.PrefetchScalarGridSpec(
        num_scalar_prefetch=1,
        grid=(2, N_ANGLES // ANG_BLK),
        in_specs=[
            pl.BlockSpec((ANG_BLK, N_CHUNK, BATCH, 128), lambda h, c, sc: (c, 0, 0, 0)),
        ],
        out_specs=[pl.BlockSpec((BATCH, X_HALF, 128), lambda h, c, sc: (0, h, 0))
                   for _ in range(4)],
        scratch_shapes=[
            pltpu.VMEM((ANG_BLK, BATCH, N_CHUNK, 8, 128), jnp.int32),
        ],
    )
    vols = pl.pallas_call(
        _bp_kernel,
        grid_spec=grid_spec,
        out_shape=[jax.ShapeDtypeStruct((BATCH, VOL_X, 128), jnp.float32)
                   for _ in range(4)],
        compiler_params=pltpu.CompilerParams(
            dimension_semantics=("parallel", "arbitrary"),
        ),
    )(sc, sino_p)

    vol = jnp.concatenate(vols, axis=-1)
    return vol[..., None] * jnp.float32(np.pi / N_ANGLES)


# speedup vs baseline: 314.1171x; 1.0029x over previous
import jax
import jax.numpy as jnp
import numpy as np
from jax.experimental import pallas as pl
from jax.experimental.pallas import tpu as pltpu

VOL_X, VOL_Y = 512, 512
N_DET = 736
N_ANGLES = 720
BATCH = 8
D_PAD = 768
N_CHUNK = D_PAD // 128
DET_CENTER = (N_DET - 1) / 2.0
ANG_BLK = 8
X_HALF = VOL_X // 2
N_IG = X_HALF // 8


def _bp_kernel(sc_ref, sino_ref, o0_ref, o1_ref, o2_ref, o3_ref, tab_ref):
    h = pl.program_id(0)
    c = pl.program_id(1)
    out_refs = (o0_ref, o1_ref, o2_ref, o3_ref)

    @pl.when(c == 0)
    def _init():
        for o in out_refs:
            o[...] = jnp.zeros((BATCH, X_HALF, 128), jnp.float32)

    for k in range(ANG_BLK):
        for ch in range(N_CHUNK):
            src = sino_ref[k, ch]
            if ch < N_CHUNK - 1:
                nxt = sino_ref[k, ch + 1][:, :1]
            else:
                nxt = jnp.zeros((BATCH, 1), jnp.float32)
            srcs = jnp.concatenate([src[:, 1:], nxt], axis=-1)
            packed = pltpu.pack_elementwise([src, srcs], packed_dtype=jnp.bfloat16)
            for b in range(BATCH):
                tab_ref[k, b, ch] = jnp.broadcast_to(packed[b:b + 1], (8, 128))

    cs = [(sc_ref[c * ANG_BLK + k, 0], sc_ref[c * ANG_BLK + k, 1])
          for k in range(ANG_BLK)]
    corner = [jnp.where(cos_a < 0.0, 7.0, 0.0).astype(jnp.float32)
              for cos_a, _ in cs]

    x_off = (h * X_HALF).astype(jnp.float32) - (VOL_X - 1) / 2.0
    xsv = jax.lax.broadcasted_iota(jnp.int32, (8, 128), 0).astype(jnp.float32) + x_off
    ysl = jax.lax.broadcasted_iota(jnp.int32, (8, 128), 1).astype(jnp.float32)

    cosv = [jnp.zeros((8, 128), jnp.float32) + cos_a for cos_a, _ in cs]
    yssin = [[(ysl + jnp.float32(128.0 * t - (VOL_Y - 1) / 2.0)) * sin_a + DET_CENTER
              for cos_a, sin_a in cs] for t in range(4)]

    def body(ig, _):
        ib = pl.multiple_of(ig * 8, 8)
        ig8 = (ig * 8).astype(jnp.float32)
        xsc = xsv + ig8
        for t in range(4):
            ct = jnp.float32(128.0 * t - (VOL_Y - 1) / 2.0)
            gathered = []
            for k in range(ANG_BLK):
                cos_a, sin_a = cs[k]
                q = ((corner[k] + x_off) + ig8) * cos_a + (ct * sin_a + DET_CENTER)
                qt = q.astype(jnp.int32)
                pbase = jnp.clip(qt, 0, N_DET - 97) & ~jnp.int32(127)
                p = pbase >> 7
                pb128 = (pbase + 128).astype(jnp.float32)

                u = xsc * cosv[k] + yssin[t][k]
                u_t = jnp.trunc(u)
                w = u - u_t
                idxm = u_t.astype(jnp.int32) & 127
                m = u < pb128
                valid = (u >= 0.0) & (u <= N_DET - 1.0)
                w1 = jnp.where(valid, w, 0.0)
                w0 = jnp.where(valid, 1.0 - w, 0.0)
                gathered.append((idxm, m, w0, w1, p))

            accs = [out_refs[t][b, pl.ds(ib, 8), :]
                    for b in range(BATCH)]
            for k in range(ANG_BLK):
                idxm, m, w0, w1, p = gathered[k]
                for b in range(BATCH):
                    word = jnp.where(m,
                                     jnp.take_along_axis(tab_ref[k, b, p], idxm, axis=1),
                                     jnp.take_along_axis(tab_ref[k, b, p + 1], idxm, axis=1))
                    g0 = pltpu.unpack_elementwise(word, index=0,
                                                  packed_dtype=jnp.bfloat16,
                                                  unpacked_dtype=jnp.float32)
                    g1 = pltpu.unpack_elementwise(word, index=1,
                                                  packed_dtype=jnp.bfloat16,
                                                  unpacked_dtype=jnp.float32)
                    accs[b] = accs[b] + (g0 * w0 + g1 * w1)
            for b in range(BATCH):
                out_refs[t][b, pl.ds(ib, 8), :] = accs[b]
        return _

    jax.lax.fori_loop(0, N_IG, body, None)


@jax.jit
def kernel(x):
    sino = jnp.transpose(x[..., 0], (1, 0, 2))
    sino_p = jnp.pad(sino, ((0, 0), (0, 0), (0, D_PAD - N_DET)))
    sino_p = sino_p.reshape(N_ANGLES, BATCH, N_CHUNK, 128).transpose(0, 2, 1, 3)

    th = np.arange(N_ANGLES, dtype=np.float64) * (np.pi / N_ANGLES)
    sc = jnp.asarray(np.stack([np.cos(th), np.sin(th)], axis=1).astype(np.float32))

    grid_spec = pltpu.PrefetchScalarGridSpec(
        num_scalar_prefetch=1,
        grid=(2, N_ANGLES // ANG_BLK),
        in_specs=[
            pl.BlockSpec((ANG_BLK, N_CHUNK, BATCH, 128), lambda h, c, sc: (c, 0, 0, 0)),
        ],
        out_specs=[pl.BlockSpec((BATCH, X_HALF, 128), lambda h, c, sc: (0, h, 0))
                   for _ in range(4)],
        scratch_shapes=[
            pltpu.VMEM((ANG_BLK, BATCH, N_CHUNK, 8, 128), jnp.int32),
        ],
    )
    vols = pl.pallas_call(
        _bp_kernel,
        grid_spec=grid_spec,
        out_shape=[jax.ShapeDtypeStruct((BATCH, VOL_X, 128), jnp.float32)
                   for _ in range(4)],
        compiler_params=pltpu.CompilerParams(
            dimension_semantics=("parallel", "arbitrary"),
        ),
    )(sc, sino_p)

    vol = jnp.concatenate(vols, axis=-1)
    return vol[..., None] * jnp.float32(np.pi / N_ANGLES)
